# baseline (speedup 1.0000x reference)
import functools

import jax
import jax.numpy as jnp
from jax.experimental import pallas as pl
from jax.experimental.pallas import tpu as pltpu


def _body(x_ref, t_ref, o_ref, ts_ref, *, scale, bt):
    i = pl.program_id(0)

    @pl.when(i == 0)
    def _():
        ts_ref[...] = jnp.transpose(t_ref[...], (1, 0)) * scale

    t = ts_ref[pl.ds(pl.multiple_of(i * bt, bt), bt), :]
    o_ref[...] = x_ref[...] + t[:, :, None]


def kernel(inputs, lookup_table):
    B, T, U = inputs.shape
    scale = float(U) ** 0.5

    x = jnp.transpose(inputs, (1, 2, 0))
    tab_t = jnp.transpose(lookup_table, (1, 0))

    BT = 8
    NB = 2
    grid = (T // BT, NB)
    out = pl.pallas_call(
        functools.partial(_body, scale=scale, bt=BT),
        grid=grid,
        in_specs=[
            pl.BlockSpec((BT, U, B // NB), lambda i, j: (i, 0, j)),
            pl.BlockSpec((U, T), lambda i, j: (0, 0)),
        ],
        out_specs=pl.BlockSpec((BT, U, B // NB), lambda i, j: (i, 0, j)),
        out_shape=jax.ShapeDtypeStruct((T, U, B), jnp.float32),
        scratch_shapes=[pltpu.VMEM((T, U), jnp.float32)],
        compiler_params=pltpu.CompilerParams(
            dimension_semantics=("arbitrary", "arbitrary"),
        ),
    )(x, tab_t)
    return jnp.transpose(out, (2, 0, 1))


# speedup vs baseline: 1.0198x; 1.0198x over previous
import functools

import jax
import jax.numpy as jnp
from jax.experimental import pallas as pl
from jax.experimental.pallas import tpu as pltpu


def _body(x_ref, t_ref, o_ref, ts_ref, *, scale, bt):
    i = pl.program_id(0)

    @pl.when(i == 0)
    def _():
        ts_ref[...] = jnp.transpose(t_ref[...], (1, 0)) * scale

    t = ts_ref[pl.ds(pl.multiple_of(i * bt, bt), bt), :]
    o_ref[...] = x_ref[...] + t[:, :, None]


def kernel(inputs, lookup_table):
    B, T, U = inputs.shape
    scale = float(U) ** 0.5

    x = jnp.transpose(inputs, (1, 2, 0))
    tab_t = jnp.transpose(lookup_table, (1, 0))

    BT = 8
    grid = (T // BT,)
    out = pl.pallas_call(
        functools.partial(_body, scale=scale, bt=BT),
        grid=grid,
        in_specs=[
            pl.BlockSpec((BT, U, B), lambda i: (i, 0, 0)),
            pl.BlockSpec((U, T), lambda i: (0, 0)),
        ],
        out_specs=pl.BlockSpec((BT, U, B), lambda i: (i, 0, 0)),
        out_shape=jax.ShapeDtypeStruct((T, U, B), jnp.float32),
        scratch_shapes=[pltpu.VMEM((T, U), jnp.float32)],
        compiler_params=pltpu.CompilerParams(
            dimension_semantics=("arbitrary",),
        ),
    )(x, tab_t)
    return jnp.transpose(out, (2, 0, 1))


# speedup vs baseline: 1.0211x; 1.0013x over previous
import functools

import jax
import jax.numpy as jnp
from jax.experimental import pallas as pl
from jax.experimental.pallas import tpu as pltpu


def _body(x_ref, t_ref, o_ref, ts_ref, *, scale, bt):
    i = pl.program_id(0)

    @pl.when(i == 0)
    def _():
        ts_ref[...] = jnp.transpose(t_ref[...], (1, 0)) * scale

    t = ts_ref[pl.ds(pl.multiple_of(i * bt, bt), bt), :]
    o_ref[...] = x_ref[...] + t[:, :, None]


def kernel(inputs, lookup_table):
    B, T, U = inputs.shape
    scale = float(U) ** 0.5

    x = jnp.transpose(inputs, (1, 2, 0))
    tab_t = jnp.transpose(lookup_table, (1, 0))

    BT = 8
    grid = (T // BT,)
    out = pl.pallas_call(
        functools.partial(_body, scale=scale, bt=BT),
        grid=grid,
        in_specs=[
            pl.BlockSpec((BT, U, B), lambda i: (i, 0, 0)),
            pl.BlockSpec((U, T), lambda i: (0, 0)),
        ],
        out_specs=pl.BlockSpec((BT, U, B), lambda i: (i, 0, 0)),
        out_shape=jax.ShapeDtypeStruct((T, U, B), jnp.float32),
        scratch_shapes=[pltpu.VMEM((T, U), jnp.float32)],
        compiler_params=pltpu.CompilerParams(
            dimension_semantics=("parallel",),
        ),
    )(x, tab_t)
    return jnp.transpose(out, (2, 0, 1))
